# baseline (speedup 1.0000x reference)
import jax
import jax.numpy as jnp
from jax.experimental import pallas as pl
from jax.experimental.pallas import tpu as pltpu

N, K, T, H = 16, 64, 64, 1024


def _body(ids_ref, ctx_ref, bc_ref, bk_ref, ckm_ref, pmask_ref,
          wc_hbm, wk_hbm, pool1_hbm, pool0_hbm,
          score_ref, enc_hbm, mask_ref, use_ref,
          wc_v, wk_v, p1_v, enc_v, gsem, ssem, wcsem, wksem, p1sem):
    copies = []
    for n in range(N):
        idn = ids_ref[n]
        c = pltpu.make_async_copy(pool0_hbm.at[n, idn], enc_v.at[n], gsem)
        c.start()
        copies.append(c)
    cwc = pltpu.make_async_copy(wc_hbm, wc_v, wcsem)
    cwk = pltpu.make_async_copy(wk_hbm, wk_v, wksem)
    cp1 = pltpu.make_async_copy(pool1_hbm, p1_v, p1sem)
    cwc.start()
    cwk.start()
    cp1.start()

    x = ctx_ref[0, :, 0, :]
    y = ctx_ref[0, :, 1, :]
    cwc.wait()
    cp = (jax.lax.dot_general(x, wc_v[:, :H], (((1,), (1,)), ((), ())),
                              preferred_element_type=jnp.float32)
          + jax.lax.dot_general(y, wc_v[:, H:], (((1,), (1,)), ((), ())),
                                preferred_element_type=jnp.float32)
          + bc_ref[...])
    cwk.wait()
    v = jax.lax.dot_general(cp, wk_v[...], (((1,), (0,)), ((), ())),
                            preferred_element_type=jnp.float32)
    sb = jnp.sum(cp * bk_ref[...], axis=1, keepdims=True)
    cp1.wait()
    p1 = p1_v[...]
    sc = jax.lax.dot_general(
        p1, v, (((2,), (1,)), ((0,), (0,))),
        preferred_element_type=jnp.float32)
    sc = sc + sb
    m = ckm_ref[...]
    sc = jnp.where(m != 0.0, sc, jnp.asarray(-1e20, jnp.float32))
    score_ref[...] = sc

    for n in range(N):
        idn = ids_ref[n]
        use_ref[pl.ds(n, 1), :] = p1_v[n, pl.ds(idn, 1), :]
        mask_ref[pl.ds(n, 1), :] = pmask_ref[n, pl.ds(idn, 1), :]

    stores = []
    for n in range(N):
        copies[n].wait()
        s = pltpu.make_async_copy(enc_v.at[n], enc_hbm.at[n], ssem)
        s.start()
        stores.append(s)
    for s in stores:
        s.wait()


def kernel(contexts_encoded, knowledge_tracking_pool_encoded_0,
           knowledge_tracking_pool_encoded_1, knowledge_tracking_pool_mask,
           tracking_ck_mask, knowledge_tracking_label, Wc, bc, Wk, bk):
    pool0 = knowledge_tracking_pool_encoded_0
    pool1 = knowledge_tracking_pool_encoded_1
    ids = knowledge_tracking_label.astype(jnp.int32)
    bc2 = bc.reshape(1, H)
    bk2 = bk.reshape(1, H)
    ckm = tracking_ck_mask.astype(jnp.float32)
    pmask = knowledge_tracking_pool_mask.astype(jnp.float32)

    grid_spec = pltpu.PrefetchScalarGridSpec(
        num_scalar_prefetch=1,
        grid=(1,),
        in_specs=[
            pl.BlockSpec((1, N, 2, H), lambda i, ids: (1, 0, 0, 0)),
            pl.BlockSpec((1, H), lambda i, ids: (0, 0)),
            pl.BlockSpec((1, H), lambda i, ids: (0, 0)),
            pl.BlockSpec((N, K), lambda i, ids: (0, 0)),
            pl.BlockSpec((N, K, T), lambda i, ids: (0, 0, 0)),
            pl.BlockSpec(memory_space=pltpu.MemorySpace.HBM),
            pl.BlockSpec(memory_space=pltpu.MemorySpace.HBM),
            pl.BlockSpec(memory_space=pltpu.MemorySpace.HBM),
            pl.BlockSpec(memory_space=pltpu.MemorySpace.HBM),
        ],
        out_specs=[
            pl.BlockSpec((N, K), lambda i, ids: (0, 0)),
            pl.BlockSpec(memory_space=pltpu.MemorySpace.HBM),
            pl.BlockSpec((N, T), lambda i, ids: (0, 0)),
            pl.BlockSpec((N, H), lambda i, ids: (0, 0)),
        ],
        scratch_shapes=[
            pltpu.VMEM((H, 2 * H), jnp.float32),
            pltpu.VMEM((H, H), jnp.float32),
            pltpu.VMEM((N, K, H), jnp.float32),
            pltpu.VMEM((N, T, H), jnp.float32),
            pltpu.SemaphoreType.DMA,
            pltpu.SemaphoreType.DMA,
            pltpu.SemaphoreType.DMA,
            pltpu.SemaphoreType.DMA,
            pltpu.SemaphoreType.DMA,
        ],
    )
    score, enc, maskf, use = pl.pallas_call(
        _body,
        grid_spec=grid_spec,
        out_shape=[
            jax.ShapeDtypeStruct((N, K), jnp.float32),
            jax.ShapeDtypeStruct((N, T, H), jnp.float32),
            jax.ShapeDtypeStruct((N, T), jnp.float32),
            jax.ShapeDtypeStruct((N, H), jnp.float32),
        ],
    )(ids, contexts_encoded, bc2, bk2, ckm, pmask, Wc, Wk, pool1, pool0)

    return (score, enc, maskf.astype(bool), use)


# speedup vs baseline: 1.0531x; 1.0531x over previous
import jax
import jax.numpy as jnp
from jax.experimental import pallas as pl
from jax.experimental.pallas import tpu as pltpu

N, K, T, H = 16, 64, 64, 1024


def _body(ids_ref, ctx_ref, bc_ref, bk_ref, ckm_ref, pmask_ref,
          wc_hbm, wk_hbm, pool1_hbm, pool0_hbm,
          score_ref, enc_ref, mask_ref, use_ref,
          wc_v, wk_v, p1_v, gsem, wcsem, wksem, p1sem):
    copies = []
    for n in range(N):
        idn = ids_ref[n]
        c = pltpu.make_async_copy(pool0_hbm.at[n, idn], enc_ref.at[n], gsem)
        c.start()
        copies.append(c)
    cwc = pltpu.make_async_copy(wc_hbm, wc_v, wcsem)
    cwk = pltpu.make_async_copy(wk_hbm, wk_v, wksem)
    cp1 = pltpu.make_async_copy(pool1_hbm, p1_v, p1sem)
    cwc.start()
    cwk.start()
    cp1.start()

    x = ctx_ref[0, :, 0, :]
    y = ctx_ref[0, :, 1, :]
    cwc.wait()
    cp = (jax.lax.dot_general(x, wc_v[:, :H], (((1,), (1,)), ((), ())),
                              preferred_element_type=jnp.float32)
          + jax.lax.dot_general(y, wc_v[:, H:], (((1,), (1,)), ((), ())),
                                preferred_element_type=jnp.float32)
          + bc_ref[...])
    cwk.wait()
    v = jax.lax.dot_general(cp, wk_v[...], (((1,), (0,)), ((), ())),
                            preferred_element_type=jnp.float32)
    sb = jnp.sum(cp * bk_ref[...], axis=1, keepdims=True)
    cp1.wait()
    p1 = p1_v[...]
    sc = jax.lax.dot_general(
        p1, v, (((2,), (1,)), ((0,), (0,))),
        preferred_element_type=jnp.float32)
    sc = sc + sb
    m = ckm_ref[...]
    sc = jnp.where(m != 0.0, sc, jnp.asarray(-1e20, jnp.float32))
    score_ref[...] = sc

    for n in range(N):
        idn = ids_ref[n]
        use_ref[pl.ds(n, 1), :] = p1_v[n, pl.ds(idn, 1), :]
        mask_ref[pl.ds(n, 1), :] = pmask_ref[n, pl.ds(idn, 1), :]

    for c in copies:
        c.wait()


def kernel(contexts_encoded, knowledge_tracking_pool_encoded_0,
           knowledge_tracking_pool_encoded_1, knowledge_tracking_pool_mask,
           tracking_ck_mask, knowledge_tracking_label, Wc, bc, Wk, bk):
    pool0 = knowledge_tracking_pool_encoded_0
    pool1 = knowledge_tracking_pool_encoded_1
    ids = knowledge_tracking_label.astype(jnp.int32)
    bc2 = bc.reshape(1, H)
    bk2 = bk.reshape(1, H)
    ckm = tracking_ck_mask.astype(jnp.float32)
    pmask = knowledge_tracking_pool_mask.astype(jnp.float32)

    grid_spec = pltpu.PrefetchScalarGridSpec(
        num_scalar_prefetch=1,
        grid=(1,),
        in_specs=[
            pl.BlockSpec((1, N, 2, H), lambda i, ids: (1, 0, 0, 0)),
            pl.BlockSpec((1, H), lambda i, ids: (0, 0)),
            pl.BlockSpec((1, H), lambda i, ids: (0, 0)),
            pl.BlockSpec((N, K), lambda i, ids: (0, 0)),
            pl.BlockSpec((N, K, T), lambda i, ids: (0, 0, 0)),
            pl.BlockSpec(memory_space=pltpu.MemorySpace.HBM),
            pl.BlockSpec(memory_space=pltpu.MemorySpace.HBM),
            pl.BlockSpec(memory_space=pltpu.MemorySpace.HBM),
            pl.BlockSpec(memory_space=pltpu.MemorySpace.HBM),
        ],
        out_specs=[
            pl.BlockSpec((N, K), lambda i, ids: (0, 0)),
            pl.BlockSpec((N, T, H), lambda i, ids: (0, 0, 0)),
            pl.BlockSpec((N, T), lambda i, ids: (0, 0)),
            pl.BlockSpec((N, H), lambda i, ids: (0, 0)),
        ],
        scratch_shapes=[
            pltpu.VMEM((H, 2 * H), jnp.float32),
            pltpu.VMEM((H, H), jnp.float32),
            pltpu.VMEM((N, K, H), jnp.float32),
            pltpu.SemaphoreType.DMA,
            pltpu.SemaphoreType.DMA,
            pltpu.SemaphoreType.DMA,
            pltpu.SemaphoreType.DMA,
        ],
    )
    score, enc, maskf, use = pl.pallas_call(
        _body,
        grid_spec=grid_spec,
        out_shape=[
            jax.ShapeDtypeStruct((N, K), jnp.float32),
            jax.ShapeDtypeStruct((N, T, H), jnp.float32),
            jax.ShapeDtypeStruct((N, T), jnp.float32),
            jax.ShapeDtypeStruct((N, H), jnp.float32),
        ],
    )(ids, contexts_encoded, bc2, bk2, ckm, pmask, Wc, Wk, pool1, pool0)

    return (score, enc, maskf.astype(bool), use)
